# baseline (speedup 1.0000x reference)
import functools

import jax
import jax.numpy as jnp
from jax import lax
from jax.experimental import pallas as pl
from jax.experimental.pallas import tpu as pltpu
from jax.experimental.pallas import tpu_sc as plsc

NUM_ROWS = 100000
DIM = 128

NC = 2
NS = 16
NW = NC * NS

B_TOTAL = 4096 * 200
B_PER_W = B_TOTAL // NW
CHUNK = 128
N_CHUNKS = B_PER_W // CHUNK
NBUF = 5
LOOK = 3
N_GROUPS = N_CHUNKS // NBUF
assert N_CHUNKS % NBUF == 0 and LOOK <= NBUF - 2


def _body(ids_hbm, w_hbm, out_hbm, idx2, rows, s0, s1, s2, s3, s4):
    sems = (s0, s1, s2, s3, s4)
    wid = lax.axis_index("s") * NC + lax.axis_index("c")
    base = wid * B_PER_W

    pltpu.sync_copy(ids_hbm.at[wid], idx2)

    def fire_gather(c, b):
        pltpu.async_copy(w_hbm.at[idx2.at[c]], rows.at[b], sems[b])

    def wait_gather(b):
        pltpu.make_async_copy(w_hbm.at[idx2.at[0]], rows.at[b], sems[b]).wait()

    def write_out(g, b):
        pltpu.sync_copy(rows.at[b], out_hbm.at[pl.ds(base + g * CHUNK, CHUNK)])

    def step(g, b):
        wait_gather(b)
        fire_gather(g + LOOK, (b + LOOK) % NBUF)
        write_out(g, b)

    for b in range(LOOK):
        fire_gather(b, b)

    def outer(go, carry):
        for b in range(NBUF):
            step(go * NBUF + b, b)
        return carry

    lax.fori_loop(0, N_GROUPS - 1, outer, 0)
    for b in range(NBUF - LOOK):
        step((N_GROUPS - 1) * NBUF + b, b)
    for b in range(NBUF - LOOK, NBUF):
        g = (N_GROUPS - 1) * NBUF + b
        wait_gather(b)
        write_out(g, b)


@jax.jit
def _run(ids3, weight):
    f = pl.kernel(
        _body,
        out_type=jax.ShapeDtypeStruct((B_TOTAL, DIM), jnp.float32),
        mesh=plsc.VectorSubcoreMesh(core_axis_name="c", subcore_axis_name="s"),
        scratch_types=[
            pltpu.VMEM((N_CHUNKS, CHUNK), jnp.int32),
            pltpu.VMEM((NBUF, CHUNK, DIM), jnp.float32),
        ] + [pltpu.SemaphoreType.DMA] * NBUF,
    )
    return f(ids3, weight)


def kernel(ids, weight):
    ids3 = ids.reshape(NW, N_CHUNKS, CHUNK).astype(jnp.int32)
    out = _run(ids3, weight)
    return out.reshape(ids.shape[0], ids.shape[1], DIM)


# speedup vs baseline: 1.0033x; 1.0033x over previous
import functools

import jax
import jax.numpy as jnp
from jax import lax
from jax.experimental import pallas as pl
from jax.experimental.pallas import tpu as pltpu
from jax.experimental.pallas import tpu_sc as plsc

NUM_ROWS = 100000
DIM = 128

NC = 2
NS = 16
NW = NC * NS

B_TOTAL = 4096 * 200
B_PER_W = B_TOTAL // NW
CHUNK = 128
N_CHUNKS = B_PER_W // CHUNK
NBUF = 5
LOOK = 3
N_GROUPS = N_CHUNKS // NBUF
assert N_CHUNKS % NBUF == 0 and LOOK <= NBUF - 2


def _body(ids_hbm, w_hbm, out_hbm, idx2, rows, s0, s1, s2, s3, s4, ws):
    sems = (s0, s1, s2, s3, s4)
    wid = lax.axis_index("s") * NC + lax.axis_index("c")
    base = wid * B_PER_W

    pltpu.sync_copy(ids_hbm.at[wid], idx2)

    def fire_gather(c, b):
        pltpu.async_copy(w_hbm.at[idx2.at[c]], rows.at[b], sems[b])

    def wait_gather(b):
        pltpu.make_async_copy(w_hbm.at[idx2.at[0]], rows.at[b], sems[b]).wait()

    def fire_write(g, b):
        pltpu.async_copy(rows.at[b], out_hbm.at[pl.ds(base + g * CHUNK, CHUNK)],
                         ws)

    def wait_write(b):
        pltpu.make_async_copy(rows.at[b], out_hbm.at[pl.ds(base, CHUNK)],
                              ws).wait()

    def step(g, b, first=False):
        if not first:
            wait_write((b - 1) % NBUF)
        wait_gather(b)
        fire_gather(g + LOOK, (b + LOOK) % NBUF)
        fire_write(g, b)

    for b in range(LOOK):
        fire_gather(b, b)

    step(0, 0, first=True)
    for g in range(1, NBUF):
        step(g, g % NBUF)

    def outer(go, carry):
        for b in range(NBUF):
            step(go * NBUF + b, b)
        return carry

    lax.fori_loop(1, N_GROUPS - 1, outer, 0)
    for b in range(NBUF - LOOK):
        step((N_GROUPS - 1) * NBUF + b, b)
    for b in range(NBUF - LOOK, NBUF):
        g = (N_GROUPS - 1) * NBUF + b
        wait_write((b - 1) % NBUF)
        wait_gather(b)
        fire_write(g, b)
    wait_write(NBUF - 1)


@jax.jit
def _run(ids3, weight):
    f = pl.kernel(
        _body,
        out_type=jax.ShapeDtypeStruct((B_TOTAL, DIM), jnp.float32),
        mesh=plsc.VectorSubcoreMesh(core_axis_name="c", subcore_axis_name="s"),
        scratch_types=[
            pltpu.VMEM((N_CHUNKS, CHUNK), jnp.int32),
            pltpu.VMEM((NBUF, CHUNK, DIM), jnp.float32),
        ] + [pltpu.SemaphoreType.DMA] * (NBUF + 1),
    )
    return f(ids3, weight)


def kernel(ids, weight):
    ids3 = ids.reshape(NW, N_CHUNKS, CHUNK).astype(jnp.int32)
    out = _run(ids3, weight)
    return out.reshape(ids.shape[0], ids.shape[1], DIM)
